# baseline (speedup 1.0000x reference)
import functools

import jax
import jax.numpy as jnp
from jax import lax
from jax.experimental import pallas as pl
from jax.experimental.pallas import tpu as pltpu
from jax.experimental.pallas import tpu_sc as plsc

N = 10000
E = 320000
D_IN = 128
H = 8
D_H = 16
HD = H * D_H
ACC_W = HD + 16

NUM_CORES = 2
NUM_SUBCORES = 16
NUM_TILES = NUM_CORES * NUM_SUBCORES
E_PER_TILE = E // NUM_TILES
CHUNK = 40
NCHUNK = E_PER_TILE // CHUNK
NPAIR = NCHUNK // 2
ROWS_PER_TILE = N // NUM_SUBCORES


def _proj_body(h_ref, w_ref, as_ref, ad_ref, z_ref, es_ref, ed_ref):
    z = jnp.dot(h_ref[...], w_ref[...], preferred_element_type=jnp.float32,
                precision=lax.Precision.HIGHEST)
    z_ref[...] = z
    es_ref[...] = jnp.dot(z, as_ref[...], preferred_element_type=jnp.float32,
                          precision=lax.Precision.HIGHEST)
    ed_ref[...] = jnp.dot(z, ad_ref[...], preferred_element_type=jnp.float32,
                          precision=lax.Precision.HIGHEST)


_PROJ_BLK = 1000


@jax.jit
def _proj(h, wf, as16, ad16):
    return pl.pallas_call(
        _proj_body,
        grid=(N // _PROJ_BLK,),
        in_specs=[
            pl.BlockSpec((_PROJ_BLK, D_IN), lambda i: (i, 0)),
            pl.BlockSpec((D_IN, HD), lambda i: (0, 0)),
            pl.BlockSpec((D_IN, 16), lambda i: (0, 0)),
            pl.BlockSpec((D_IN, 16), lambda i: (0, 0)),
        ],
        out_specs=[
            pl.BlockSpec((_PROJ_BLK, HD), lambda i: (i, 0)),
            pl.BlockSpec((_PROJ_BLK, 16), lambda i: (i, 0)),
            pl.BlockSpec((_PROJ_BLK, 16), lambda i: (i, 0)),
        ],
        out_shape=[
            jax.ShapeDtypeStruct((N, HD), jnp.float32),
            jax.ShapeDtypeStruct((N, 16), jnp.float32),
            jax.ShapeDtypeStruct((N, 16), jnp.float32),
        ],
    )(h, wf, as16, ad16)


def _edge_body(src_hbm, dst_hbm, z_hbm, es_hbm, ed_hbm, zero_hbm, out_hbm,
               si0, si1, didx, zb0, zb1, eb0, eb1, db0, db1, mb0, mb1,
               acc, is0, is1, gs0, gs1, ss0, ss1):
    c = lax.axis_index("c")
    s = lax.axis_index("s")
    tid = c * NUM_SUBCORES + s
    r0 = s * ROWS_PER_TILE

    pltpu.sync_copy(zero_hbm, acc.at[pl.ds(r0, ROWS_PER_TILE)])
    pltpu.sync_copy(dst_hbm.at[pl.ds(tid * NCHUNK, NCHUNK)], didx)
    plsc.subcore_barrier()

    iota = lax.broadcasted_iota(jnp.int32, (16,), 0)
    mask_hi = jnp.where(iota >= 8, 1.0, 0.0).astype(jnp.float32)

    sidxs = [si0, si1]
    zbufs = [zb0, zb1]
    esbufs = [eb0, eb1]
    edbufs = [db0, db1]
    msgbufs = [mb0, mb1]
    isems = [is0, is1]
    gsems = [gs0, gs1]
    ssems = [ss0, ss1]

    def issue_sidx(ci, b):
        pltpu.async_copy(src_hbm.at[pl.ds(tid * E_PER_TILE + ci * CHUNK,
                                          CHUNK)],
                         sidxs[b], isems[b])

    def wait_sidx(b):
        pltpu.make_async_copy(src_hbm.at[pl.ds(0, CHUNK)], sidxs[b],
                              isems[b]).wait()

    def issue_gathers(ci, b):
        pltpu.async_copy(z_hbm.at[sidxs[b]], zbufs[b], gsems[b])
        pltpu.async_copy(es_hbm.at[sidxs[b]], esbufs[b], gsems[b])
        pltpu.async_copy(ed_hbm.at[didx.at[ci]], edbufs[b], gsems[b])

    def wait_gathers(b):
        pltpu.make_async_copy(z_hbm.at[sidxs[b]], zbufs[b], gsems[b]).wait()
        pltpu.make_async_copy(es_hbm.at[sidxs[b]], esbufs[b],
                              gsems[b]).wait()
        pltpu.make_async_copy(ed_hbm.at[didx.at[0]], edbufs[b],
                              gsems[b]).wait()

    def wait_scatter(b):
        pltpu.make_async_copy(msgbufs[b], acc.at[didx.at[0]], ssems[b]).wait()

    issue_sidx(0, 0)
    issue_sidx(1, 1)
    wait_sidx(0)
    issue_gathers(0, 0)

    def pair_body(i, carry):
        for b in range(2):
            ci = 2 * i + b
            wait_gathers(b)

            if b == 0:
                wait_sidx(1)
                issue_gathers(ci + 1, 1)
            else:
                @pl.when(i < NPAIR - 1)
                def _():
                    wait_sidx(0)
                    issue_gathers(ci + 1, 0)

            @pl.when(i < NPAIR - 1)
            def _():
                issue_sidx(ci + 2, b)

            @pl.when(i > 0)
            def _():
                wait_scatter(b)

            zbuf, esbuf, edbuf, msgbuf = (
                zbufs[b], esbufs[b], edbufs[b], msgbufs[b])

            @plsc.parallel_loop(0, CHUNK, unroll=8)
            def _(j):
                sm = esbuf[j] + edbuf[j]
                sm = jnp.where(sm > 0, sm, 0.2 * sm)
                w = jnp.exp(sm) * mask_hi
                for hh in range(H):
                    wh = lax.gather(
                        w, jnp.full((16, 1), 8 + hh, jnp.int32),
                        dimension_numbers=lax.GatherDimensionNumbers(
                            offset_dims=(), collapsed_slice_dims=(0,),
                            start_index_map=(0,)),
                        slice_sizes=(1,),
                        mode=lax.GatherScatterMode.PROMISE_IN_BOUNDS)
                    msgbuf[j, pl.ds(hh * D_H, D_H)] = (
                        wh * zbuf[j, pl.ds(hh * D_H, D_H)])
                msgbuf[j, pl.ds(HD, 16)] = w
            pltpu.async_copy(msgbufs[b], acc.at[didx.at[ci]], ssems[b],
                             add=True)
        return carry

    lax.fori_loop(0, NPAIR, pair_body, 0)
    wait_scatter(0)
    wait_scatter(1)
    plsc.subcore_barrier()
    pltpu.sync_copy(acc.at[pl.ds(r0, ROWS_PER_TILE)],
                    out_hbm.at[c, pl.ds(r0, ROWS_PER_TILE)])


@jax.jit
def _edge(src, dst, z, es16, ed16, zero):
    mesh = plsc.VectorSubcoreMesh(core_axis_name="c", subcore_axis_name="s")
    return pl.kernel(
        _edge_body,
        out_type=jax.ShapeDtypeStruct((NUM_CORES, N, ACC_W), jnp.float32),
        mesh=mesh,
        scratch_types=[
            pltpu.VMEM((CHUNK,), jnp.int32),
            pltpu.VMEM((CHUNK,), jnp.int32),
            pltpu.VMEM((NCHUNK, CHUNK), jnp.int32),
            pltpu.VMEM((CHUNK, HD), jnp.float32),
            pltpu.VMEM((CHUNK, HD), jnp.float32),
            pltpu.VMEM((CHUNK, 16), jnp.float32),
            pltpu.VMEM((CHUNK, 16), jnp.float32),
            pltpu.VMEM((CHUNK, 16), jnp.float32),
            pltpu.VMEM((CHUNK, 16), jnp.float32),
            pltpu.VMEM((CHUNK, ACC_W), jnp.float32),
            pltpu.VMEM((CHUNK, ACC_W), jnp.float32),
            pltpu.VMEM_SHARED((N, ACC_W), jnp.float32),
            pltpu.SemaphoreType.DMA,
            pltpu.SemaphoreType.DMA,
            pltpu.SemaphoreType.DMA,
            pltpu.SemaphoreType.DMA,
            pltpu.SemaphoreType.DMA,
            pltpu.SemaphoreType.DMA,
        ],
        compiler_params=pltpu.CompilerParams(
            use_tc_tiling_on_sc=False, needs_layout_passes=False),
    )(src, dst.reshape(NUM_TILES * NCHUNK, CHUNK),
      z, es16, ed16, zero)


def _final_body(acc_ref, o_ref, r8_ref, out_ref):
    a = acc_ref[0] + acc_ref[1]
    att = a[:, :HD]
    den = a[:, HD + 8:HD + 16]
    denrep = jnp.dot(den, r8_ref[...], preferred_element_type=jnp.float32,
                     precision=lax.Precision.HIGHEST)
    x = att / (denrep + 1e-9)
    out_ref[...] = o_ref[...] + jnp.where(x > 0, x, jnp.exp(jnp.minimum(x, 0.0)) - 1.0)


_FIN_BLK = 1000


@jax.jit
def _final(accs, o, r8):
    return pl.pallas_call(
        _final_body,
        grid=(N // _FIN_BLK,),
        in_specs=[
            pl.BlockSpec((NUM_CORES, _FIN_BLK, ACC_W), lambda i: (0, i, 0)),
            pl.BlockSpec((_FIN_BLK, HD), lambda i: (i, 0)),
            pl.BlockSpec((H, HD), lambda i: (0, 0)),
        ],
        out_specs=pl.BlockSpec((_FIN_BLK, HD), lambda i: (i, 0)),
        out_shape=jax.ShapeDtypeStruct((N, HD), jnp.float32),
    )(accs, o, r8)


def kernel(edge_index, o, h, W, a_src, a_dst):
    src = edge_index[0].astype(jnp.int32)
    dst = edge_index[1].astype(jnp.int32)
    wf = W.reshape(D_IN, HD).astype(jnp.float32)
    cols = jnp.arange(HD, dtype=jnp.int32)
    as16 = jnp.zeros((HD, 16), jnp.float32).at[cols, 8 + cols // D_H].set(
        a_src.reshape(HD))
    ad16 = jnp.zeros((HD, 16), jnp.float32).at[cols, 8 + cols // D_H].set(
        a_dst.reshape(HD))
    r8 = jnp.zeros((H, HD), jnp.float32).at[cols // D_H, cols].set(1.0)

    z, es16, ed16 = _proj(h, wf, as16, ad16)
    zero = jnp.zeros((ROWS_PER_TILE, ACC_W), jnp.float32)
    accs = _edge(src, dst, z, es16, ed16, zero)
    return _final(accs, o, r8)


# speedup vs baseline: 1.0266x; 1.0266x over previous
import jax
import jax.numpy as jnp
from jax import lax
from jax.experimental import pallas as pl
from jax.experimental.pallas import tpu as pltpu
from jax.experimental.pallas import tpu_sc as plsc

N = 10000
E = 320000
D_IN = 128
H = 8
D_H = 16
HD = H * D_H
ACC_W = HD + 16

NUM_CORES = 2
NUM_SUBCORES = 16
NUM_TILES = NUM_CORES * NUM_SUBCORES
E_PER_TILE = E // NUM_TILES
CHUNK = 40
NCHUNK = E_PER_TILE // CHUNK
NPAIR = NCHUNK // 2
ROWS_PER_TILE = N // NUM_SUBCORES


def _proj_body(h_ref, w_ref, as_ref, ad_ref, zt_ref, ed_ref):
    z = jnp.dot(h_ref[...], w_ref[...], preferred_element_type=jnp.float32,
                precision=lax.Precision.HIGHEST)
    es = jnp.dot(z, as_ref[...], preferred_element_type=jnp.float32,
                 precision=lax.Precision.HIGHEST)
    zt_ref[...] = jnp.concatenate([z, es], axis=1)
    ed_ref[...] = jnp.dot(z, ad_ref[...], preferred_element_type=jnp.float32,
                          precision=lax.Precision.HIGHEST)


_PROJ_BLK = 1000


@jax.jit
def _proj(h, wf, as16, ad16):
    return pl.pallas_call(
        _proj_body,
        grid=(N // _PROJ_BLK,),
        in_specs=[
            pl.BlockSpec((_PROJ_BLK, D_IN), lambda i: (i, 0)),
            pl.BlockSpec((D_IN, HD), lambda i: (0, 0)),
            pl.BlockSpec((D_IN, 16), lambda i: (0, 0)),
            pl.BlockSpec((D_IN, 16), lambda i: (0, 0)),
        ],
        out_specs=[
            pl.BlockSpec((_PROJ_BLK, ACC_W), lambda i: (i, 0)),
            pl.BlockSpec((_PROJ_BLK, 16), lambda i: (i, 0)),
        ],
        out_shape=[
            jax.ShapeDtypeStruct((N, ACC_W), jnp.float32),
            jax.ShapeDtypeStruct((N, 16), jnp.float32),
        ],
    )(h, wf, as16, ad16)


def _edge_body(src_hbm, dst_hbm, zt_hbm, ed_hbm, zero_hbm, att_hbm, den_hbm,
               si0, si1, didx, zb0, zb1, db0, db1, mb0, mb1,
               acc, is0, is1, gs0, gs1, ss0, ss1):
    c = lax.axis_index("c")
    s = lax.axis_index("s")
    tid = c * NUM_SUBCORES + s
    r0 = s * ROWS_PER_TILE

    sidxs = [si0, si1]
    zbufs = [zb0, zb1]
    edbufs = [db0, db1]
    msgbufs = [mb0, mb1]
    isems = [is0, is1]
    gsems = [gs0, gs1]
    ssems = [ss0, ss1]

    def issue_sidx(ci, b):
        pltpu.async_copy(src_hbm.at[pl.ds(tid * E_PER_TILE + ci * CHUNK,
                                          CHUNK)],
                         sidxs[b], isems[b])

    def wait_sidx(b):
        pltpu.make_async_copy(src_hbm.at[pl.ds(0, CHUNK)], sidxs[b],
                              isems[b]).wait()

    def issue_gathers(ci, b):
        pltpu.async_copy(zt_hbm.at[sidxs[b]], zbufs[b], gsems[b])
        pltpu.async_copy(ed_hbm.at[didx.at[ci]], edbufs[b], gsems[b])

    def wait_gathers(b):
        pltpu.make_async_copy(zt_hbm.at[sidxs[b]], zbufs[b], gsems[b]).wait()
        pltpu.make_async_copy(ed_hbm.at[didx.at[0]], edbufs[b],
                              gsems[b]).wait()

    def wait_scatter(b):
        pltpu.make_async_copy(msgbufs[b], acc.at[didx.at[0]], ssems[b]).wait()

    issue_sidx(0, 0)
    issue_sidx(1, 1)
    pltpu.sync_copy(dst_hbm.at[pl.ds(tid * NCHUNK, NCHUNK)], didx)
    pltpu.sync_copy(zero_hbm, acc.at[pl.ds(r0, ROWS_PER_TILE)])
    wait_sidx(0)
    issue_gathers(0, 0)
    plsc.subcore_barrier()

    iota = lax.broadcasted_iota(jnp.int32, (16,), 0)
    mask_hi = jnp.where(iota >= 8, 1.0, 0.0).astype(jnp.float32)

    def pair_body(i, carry):
        for b in range(2):
            ci = 2 * i + b
            wait_gathers(b)

            if b == 0:
                wait_sidx(1)
                issue_gathers(ci + 1, 1)
            else:
                @pl.when(i < NPAIR - 1)
                def _():
                    wait_sidx(0)
                    issue_gathers(ci + 1, 0)

            @pl.when(i < NPAIR - 1)
            def _():
                issue_sidx(ci + 2, b)

            @pl.when(i > 0)
            def _():
                wait_scatter(b)

            zbuf, edbuf, msgbuf = zbufs[b], edbufs[b], msgbufs[b]

            @plsc.parallel_loop(0, CHUNK, unroll=8)
            def _(j):
                sm = zbuf[j, pl.ds(HD, 16)] + edbuf[j]
                sm = jnp.where(sm > 0, sm, 0.2 * sm)
                w = jnp.exp(sm) * mask_hi
                for hh in range(H):
                    wh = lax.gather(
                        w, jnp.full((16, 1), 8 + hh, jnp.int32),
                        dimension_numbers=lax.GatherDimensionNumbers(
                            offset_dims=(), collapsed_slice_dims=(0,),
                            start_index_map=(0,)),
                        slice_sizes=(1,),
                        mode=lax.GatherScatterMode.PROMISE_IN_BOUNDS)
                    msgbuf[j, pl.ds(hh * D_H, D_H)] = (
                        wh * zbuf[j, pl.ds(hh * D_H, D_H)])
                msgbuf[j, pl.ds(HD, 16)] = w

            pltpu.async_copy(msgbufs[b], acc.at[didx.at[ci]], ssems[b],
                             add=True)
        return carry

    lax.fori_loop(0, NPAIR, pair_body, 0)
    wait_scatter(0)
    wait_scatter(1)
    plsc.subcore_barrier()
    pltpu.sync_copy(acc.at[pl.ds(r0, ROWS_PER_TILE), pl.ds(0, HD)],
                    att_hbm.at[c, pl.ds(r0, ROWS_PER_TILE)])
    pltpu.sync_copy(acc.at[pl.ds(r0, ROWS_PER_TILE), pl.ds(HD, 16)],
                    den_hbm.at[c, pl.ds(r0, ROWS_PER_TILE)])


@jax.jit
def _edge(src, dst, ztab, ed16, zero):
    mesh = plsc.VectorSubcoreMesh(core_axis_name="c", subcore_axis_name="s")
    return pl.kernel(
        _edge_body,
        out_type=(jax.ShapeDtypeStruct((NUM_CORES, N, HD), jnp.float32),
                  jax.ShapeDtypeStruct((NUM_CORES, N, 16), jnp.float32)),
        mesh=mesh,
        scratch_types=[
            pltpu.VMEM((CHUNK,), jnp.int32),
            pltpu.VMEM((CHUNK,), jnp.int32),
            pltpu.VMEM((NCHUNK, CHUNK), jnp.int32),
            pltpu.VMEM((CHUNK, ACC_W), jnp.float32),
            pltpu.VMEM((CHUNK, ACC_W), jnp.float32),
            pltpu.VMEM((CHUNK, 16), jnp.float32),
            pltpu.VMEM((CHUNK, 16), jnp.float32),
            pltpu.VMEM((CHUNK, ACC_W), jnp.float32),
            pltpu.VMEM((CHUNK, ACC_W), jnp.float32),
            pltpu.VMEM_SHARED((N, ACC_W), jnp.float32),
            pltpu.SemaphoreType.DMA,
            pltpu.SemaphoreType.DMA,
            pltpu.SemaphoreType.DMA,
            pltpu.SemaphoreType.DMA,
            pltpu.SemaphoreType.DMA,
            pltpu.SemaphoreType.DMA,
        ],
        compiler_params=pltpu.CompilerParams(
            use_tc_tiling_on_sc=False, needs_layout_passes=False),
    )(src, dst.reshape(NUM_TILES * NCHUNK, CHUNK), ztab, ed16, zero)


def _final_body(att_ref, den_ref, o_ref, r8_ref, out_ref):
    a = att_ref[0] + att_ref[1]
    d = den_ref[0] + den_ref[1]
    den = d[:, 8:16]
    denrep = jnp.dot(den, r8_ref[...], preferred_element_type=jnp.float32,
                     precision=lax.Precision.HIGHEST)
    x = a / (denrep + 1e-9)
    out_ref[...] = o_ref[...] + jnp.where(
        x > 0, x, jnp.exp(jnp.minimum(x, 0.0)) - 1.0)


_FIN_BLK = 1000


@jax.jit
def _final(att, den, o, r8):
    return pl.pallas_call(
        _final_body,
        grid=(N // _FIN_BLK,),
        in_specs=[
            pl.BlockSpec((NUM_CORES, _FIN_BLK, HD), lambda i: (0, i, 0)),
            pl.BlockSpec((NUM_CORES, _FIN_BLK, 16), lambda i: (0, i, 0)),
            pl.BlockSpec((_FIN_BLK, HD), lambda i: (i, 0)),
            pl.BlockSpec((H, HD), lambda i: (0, 0)),
        ],
        out_specs=pl.BlockSpec((_FIN_BLK, HD), lambda i: (i, 0)),
        out_shape=jax.ShapeDtypeStruct((N, HD), jnp.float32),
    )(att, den, o, r8)


def kernel(edge_index, o, h, W, a_src, a_dst):
    src = edge_index[0].astype(jnp.int32)
    dst = edge_index[1].astype(jnp.int32)
    wf = W.reshape(D_IN, HD).astype(jnp.float32)
    cols = jnp.arange(HD, dtype=jnp.int32)
    as16 = jnp.zeros((HD, 16), jnp.float32).at[cols, 8 + cols // D_H].set(
        a_src.reshape(HD))
    ad16 = jnp.zeros((HD, 16), jnp.float32).at[cols, 8 + cols // D_H].set(
        a_dst.reshape(HD))
    r8 = jnp.zeros((H, HD), jnp.float32).at[cols // D_H, cols].set(1.0)

    ztab, ed16 = _proj(h, wf, as16, ad16)
    zero = jnp.zeros((ROWS_PER_TILE, ACC_W), jnp.float32)
    att, den = _edge(src, dst, ztab, ed16, zero)
    return _final(att, den, o, r8)


# speedup vs baseline: 1.1095x; 1.0807x over previous
import jax
import jax.numpy as jnp
from jax import lax
from jax.experimental import pallas as pl
from jax.experimental.pallas import tpu as pltpu
from jax.experimental.pallas import tpu_sc as plsc

N = 10000
E = 320000
D_IN = 128
H = 8
D_H = 16
HD = H * D_H
ACC_W = HD + 16
ZT_W = HD + 32

NUM_CORES = 2
NUM_SUBCORES = 16
NUM_TILES = NUM_CORES * NUM_SUBCORES
E_PER_TILE = E // NUM_TILES
CHUNK = 40
NCHUNK = E_PER_TILE // CHUNK
NPAIR = NCHUNK // 2
ROWS_PER_TILE = N // NUM_SUBCORES


def _proj_body(h_ref, w_ref, as_ref, ad_ref, zt_ref, ed_ref):
    z = jnp.dot(h_ref[...], w_ref[...], preferred_element_type=jnp.float32,
                precision=lax.Precision.HIGHEST)
    es = jnp.dot(z, as_ref[...], preferred_element_type=jnp.float32,
                 precision=lax.Precision.HIGHEST)
    zt_ref[...] = jnp.concatenate([z, es], axis=1).astype(jnp.bfloat16)
    ed_ref[...] = jnp.dot(z, ad_ref[...], preferred_element_type=jnp.float32,
                          precision=lax.Precision.HIGHEST)


_PROJ_BLK = 1000


@jax.jit
def _proj(h, wf, as16, ad16):
    return pl.pallas_call(
        _proj_body,
        grid=(N // _PROJ_BLK,),
        in_specs=[
            pl.BlockSpec((_PROJ_BLK, D_IN), lambda i: (i, 0)),
            pl.BlockSpec((D_IN, HD), lambda i: (0, 0)),
            pl.BlockSpec((D_IN, 32), lambda i: (0, 0)),
            pl.BlockSpec((D_IN, 16), lambda i: (0, 0)),
        ],
        out_specs=[
            pl.BlockSpec((_PROJ_BLK, ZT_W), lambda i: (i, 0)),
            pl.BlockSpec((_PROJ_BLK, 16), lambda i: (i, 0)),
        ],
        out_shape=[
            jax.ShapeDtypeStruct((N, ZT_W), jnp.bfloat16),
            jax.ShapeDtypeStruct((N, 16), jnp.float32),
        ],
    )(h, wf, as16, ad16)


def _edge_body(src_hbm, dst_hbm, zt_hbm, ed_hbm, zero_hbm, att_hbm, den_hbm,
               si0, si1, didx, zb0, zb1, db0, db1, mb0, mb1,
               acc, is0, is1, gs0, gs1, ss0, ss1):
    c = lax.axis_index("c")
    s = lax.axis_index("s")
    tid = c * NUM_SUBCORES + s
    r0 = s * ROWS_PER_TILE

    sidxs = [si0, si1]
    zbufs = [zb0, zb1]
    edbufs = [db0, db1]
    msgbufs = [mb0, mb1]
    isems = [is0, is1]
    gsems = [gs0, gs1]
    ssems = [ss0, ss1]

    def issue_sidx(ci, b):
        pltpu.async_copy(src_hbm.at[pl.ds(tid * E_PER_TILE + ci * CHUNK,
                                          CHUNK)],
                         sidxs[b], isems[b])

    def wait_sidx(b):
        pltpu.make_async_copy(src_hbm.at[pl.ds(0, CHUNK)], sidxs[b],
                              isems[b]).wait()

    def issue_gathers(ci, b):
        pltpu.async_copy(zt_hbm.at[sidxs[b]], zbufs[b], gsems[b])
        pltpu.async_copy(ed_hbm.at[didx.at[ci]], edbufs[b], gsems[b])

    def wait_gathers(b):
        pltpu.make_async_copy(zt_hbm.at[sidxs[b]], zbufs[b], gsems[b]).wait()
        pltpu.make_async_copy(ed_hbm.at[didx.at[0]], edbufs[b],
                              gsems[b]).wait()

    def wait_scatter(b):
        pltpu.make_async_copy(msgbufs[b], acc.at[didx.at[0]], ssems[b]).wait()

    issue_sidx(0, 0)
    issue_sidx(1, 1)
    pltpu.sync_copy(dst_hbm.at[pl.ds(tid * NCHUNK, NCHUNK)], didx)
    pltpu.sync_copy(zero_hbm, acc.at[pl.ds(r0, ROWS_PER_TILE)])
    wait_sidx(0)
    issue_gathers(0, 0)
    plsc.subcore_barrier()

    iota = lax.broadcasted_iota(jnp.int32, (16,), 0)
    mask_hi = jnp.where(iota >= 8, 1.0, 0.0).astype(jnp.float32)

    def pair_body(i, carry):
        for b in range(2):
            ci = 2 * i + b
            wait_gathers(b)

            if b == 0:
                wait_sidx(1)
                issue_gathers(ci + 1, 1)
            else:
                @pl.when(i < NPAIR - 1)
                def _():
                    wait_sidx(0)
                    issue_gathers(ci + 1, 0)

            @pl.when(i < NPAIR - 1)
            def _():
                issue_sidx(ci + 2, b)

            @pl.when(i > 0)
            def _():
                wait_scatter(b)

            zbuf, edbuf, msgbuf = zbufs[b], edbufs[b], msgbufs[b]

            def bcast(w, lane):
                return lax.gather(
                    w, jnp.full((16, 1), lane, jnp.int32),
                    dimension_numbers=lax.GatherDimensionNumbers(
                        offset_dims=(), collapsed_slice_dims=(0,),
                        start_index_map=(0,)),
                    slice_sizes=(1,),
                    mode=lax.GatherScatterMode.PROMISE_IN_BOUNDS)

            @plsc.parallel_loop(0, CHUNK, unroll=8)
            def _(j):
                ea, _ = plsc.unpack(zbuf[j, pl.ds(HD, 32)],
                                    format=plsc.PackFormat.INTERLEAVED)
                sm = ea + edbuf[j]
                sm = jnp.where(sm > 0, sm, 0.2 * sm)
                w = jnp.exp(sm) * mask_hi
                for k in range(4):
                    za, zb = plsc.unpack(zbuf[j, pl.ds(32 * k, 32)],
                                         format=plsc.PackFormat.INTERLEAVED)
                    msgbuf[j, pl.ds(32 * k, 16)] = bcast(w, 8 + 2 * k) * za
                    msgbuf[j, pl.ds(32 * k + 16, 16)] = (
                        bcast(w, 9 + 2 * k) * zb)
                msgbuf[j, pl.ds(HD, 16)] = w

            pltpu.async_copy(msgbufs[b], acc.at[didx.at[ci]], ssems[b],
                             add=True)
        return carry

    lax.fori_loop(0, NPAIR, pair_body, 0)
    wait_scatter(0)
    wait_scatter(1)
    plsc.subcore_barrier()
    pltpu.sync_copy(acc.at[pl.ds(r0, ROWS_PER_TILE), pl.ds(0, HD)],
                    att_hbm.at[c, pl.ds(r0, ROWS_PER_TILE)])
    pltpu.sync_copy(acc.at[pl.ds(r0, ROWS_PER_TILE), pl.ds(HD, 16)],
                    den_hbm.at[c, pl.ds(r0, ROWS_PER_TILE)])


@jax.jit
def _edge(src, dst, ztab, ed16, zero):
    mesh = plsc.VectorSubcoreMesh(core_axis_name="c", subcore_axis_name="s")
    return pl.kernel(
        _edge_body,
        out_type=(jax.ShapeDtypeStruct((NUM_CORES, N, HD), jnp.float32),
                  jax.ShapeDtypeStruct((NUM_CORES, N, 16), jnp.float32)),
        mesh=mesh,
        scratch_types=[
            pltpu.VMEM((CHUNK,), jnp.int32),
            pltpu.VMEM((CHUNK,), jnp.int32),
            pltpu.VMEM((NCHUNK, CHUNK), jnp.int32),
            pltpu.VMEM((CHUNK, ZT_W), jnp.bfloat16),
            pltpu.VMEM((CHUNK, ZT_W), jnp.bfloat16),
            pltpu.VMEM((CHUNK, 16), jnp.float32),
            pltpu.VMEM((CHUNK, 16), jnp.float32),
            pltpu.VMEM((CHUNK, ACC_W), jnp.float32),
            pltpu.VMEM((CHUNK, ACC_W), jnp.float32),
            pltpu.VMEM_SHARED((N, ACC_W), jnp.float32),
            pltpu.SemaphoreType.DMA,
            pltpu.SemaphoreType.DMA,
            pltpu.SemaphoreType.DMA,
            pltpu.SemaphoreType.DMA,
            pltpu.SemaphoreType.DMA,
            pltpu.SemaphoreType.DMA,
        ],
        compiler_params=pltpu.CompilerParams(
            use_tc_tiling_on_sc=False, needs_layout_passes=False),
    )(src, dst.reshape(NUM_TILES * NCHUNK, CHUNK), ztab, ed16, zero)


def _final_body(att_ref, den_ref, o_ref, r8_ref, out_ref):
    a = att_ref[0] + att_ref[1]
    d = den_ref[0] + den_ref[1]
    den = d[:, 8:16]
    denrep = jnp.dot(den, r8_ref[...], preferred_element_type=jnp.float32,
                     precision=lax.Precision.HIGHEST)
    x = a / (denrep + 1e-9)
    out_ref[...] = o_ref[...] + jnp.where(
        x > 0, x, jnp.exp(jnp.minimum(x, 0.0)) - 1.0)


_FIN_BLK = 1000


@jax.jit
def _final(att, den, o, r8):
    return pl.pallas_call(
        _final_body,
        grid=(N // _FIN_BLK,),
        in_specs=[
            pl.BlockSpec((NUM_CORES, _FIN_BLK, HD), lambda i: (0, i, 0)),
            pl.BlockSpec((NUM_CORES, _FIN_BLK, 16), lambda i: (0, i, 0)),
            pl.BlockSpec((_FIN_BLK, HD), lambda i: (i, 0)),
            pl.BlockSpec((H, HD), lambda i: (0, 0)),
        ],
        out_specs=pl.BlockSpec((_FIN_BLK, HD), lambda i: (i, 0)),
        out_shape=jax.ShapeDtypeStruct((N, HD), jnp.float32),
    )(att, den, o, r8)


def kernel(edge_index, o, h, W, a_src, a_dst):
    src = edge_index[0].astype(jnp.int32)
    dst = edge_index[1].astype(jnp.int32)
    cols = jnp.arange(HD, dtype=jnp.int32)
    perm = 32 * (cols // 32) + (cols % 2) * 16 + (cols % 32) // 2
    wf = W.reshape(D_IN, HD).astype(jnp.float32)[:, perm]
    af = a_src.reshape(HD)
    as32 = jnp.zeros((HD, 32), jnp.float32).at[
        cols, 16 + 2 * (perm // D_H)].set(af[perm])
    ad16 = jnp.zeros((HD, 16), jnp.float32).at[cols, 8 + cols // D_H].set(
        a_dst.reshape(HD))
    ad16 = ad16[perm, :]
    r8 = jnp.zeros((H, HD), jnp.float32).at[cols // D_H, cols].set(1.0)

    ztab, ed16 = _proj(h, wf, as32, ad16)
    zero = jnp.zeros((ROWS_PER_TILE, ACC_W), jnp.float32)
    att, den = _edge(src, dst, ztab, ed16, zero)
    return _final(att, den, o, r8)


# speedup vs baseline: 1.3982x; 1.2602x over previous
import jax
import jax.numpy as jnp
from jax import lax
from jax.experimental import pallas as pl
from jax.experimental.pallas import tpu as pltpu
from jax.experimental.pallas import tpu_sc as plsc

N = 10000
E = 320000
D_IN = 128
H = 8
D_H = 16
HD = H * D_H
ACC_W = HD + 16
ZT_W = HD + 32

NUM_CORES = 2
NUM_SUBCORES = 16
NUM_TILES = NUM_CORES * NUM_SUBCORES
E_PER_TILE = E // NUM_TILES
CHUNK = 80
NCHUNK = E_PER_TILE // CHUNK
NPAIR = NCHUNK // 2
ROWS_PER_TILE = N // NUM_SUBCORES


def _proj_body(h_ref, w_ref, as_ref, ad_ref, zt_ref, ed_ref):
    z = jnp.dot(h_ref[...], w_ref[...], preferred_element_type=jnp.float32,
                precision=lax.Precision.HIGHEST)
    es = jnp.dot(z, as_ref[...], preferred_element_type=jnp.float32,
                 precision=lax.Precision.HIGHEST)
    zt_ref[...] = jnp.concatenate([z, es], axis=1).astype(jnp.bfloat16)
    ed_ref[...] = jnp.dot(z, ad_ref[...], preferred_element_type=jnp.float32,
                          precision=lax.Precision.HIGHEST)


_PROJ_BLK = 1000


@jax.jit
def _proj(h, wf, as16, ad16):
    return pl.pallas_call(
        _proj_body,
        grid=(N // _PROJ_BLK,),
        in_specs=[
            pl.BlockSpec((_PROJ_BLK, D_IN), lambda i: (i, 0)),
            pl.BlockSpec((D_IN, HD), lambda i: (0, 0)),
            pl.BlockSpec((D_IN, 32), lambda i: (0, 0)),
            pl.BlockSpec((D_IN, 16), lambda i: (0, 0)),
        ],
        out_specs=[
            pl.BlockSpec((_PROJ_BLK, ZT_W), lambda i: (i, 0)),
            pl.BlockSpec((_PROJ_BLK, 16), lambda i: (i, 0)),
        ],
        out_shape=[
            jax.ShapeDtypeStruct((N, ZT_W), jnp.bfloat16),
            jax.ShapeDtypeStruct((N, 16), jnp.float32),
        ],
    )(h, wf, as16, ad16)


def _edge_body(src_hbm, dst_hbm, zt_hbm, ed_hbm, zero_hbm, att_hbm, den_hbm,
               si0, si1, dr0, dr1, dr2, dr3, zb0, zb1, db0, db1, mb0, mb1,
               acc, is0, is1, gs0, gs1, ss0, ss1):
    c = lax.axis_index("c")
    s = lax.axis_index("s")
    tid = c * NUM_SUBCORES + s
    r0 = s * ROWS_PER_TILE

    sidxs = [si0, si1]
    dring = [dr0, dr1, dr2, dr3]
    zbufs = [zb0, zb1]
    edbufs = [db0, db1]
    msgbufs = [mb0, mb1]
    isems = [is0, is1]
    gsems = [gs0, gs1]
    ssems = [ss0, ss1]

    def issue_idx(ci, b, dslot):
        base = tid * E_PER_TILE + ci * CHUNK
        pltpu.async_copy(src_hbm.at[pl.ds(base, CHUNK)], sidxs[b], isems[b])
        pltpu.async_copy(dst_hbm.at[pl.ds(base, CHUNK)], dring[dslot],
                         isems[b])

    def wait_idx(b):
        pltpu.make_async_copy(src_hbm.at[pl.ds(0, CHUNK)], sidxs[b],
                              isems[b]).wait()
        pltpu.make_async_copy(dst_hbm.at[pl.ds(0, CHUNK)], dring[0],
                              isems[b]).wait()

    def issue_gathers(b, dslot):
        pltpu.async_copy(zt_hbm.at[sidxs[b]], zbufs[b], gsems[b])
        pltpu.async_copy(ed_hbm.at[dring[dslot]], edbufs[b], gsems[b])

    def wait_gathers(b):
        pltpu.make_async_copy(zt_hbm.at[sidxs[b]], zbufs[b], gsems[b]).wait()
        pltpu.make_async_copy(ed_hbm.at[dring[0]], edbufs[b],
                              gsems[b]).wait()

    def wait_scatter(b):
        pltpu.make_async_copy(msgbufs[b], acc.at[dring[0]], ssems[b]).wait()

    issue_idx(0, 0, 0)
    issue_idx(1, 1, 1)
    pltpu.sync_copy(zero_hbm, acc.at[pl.ds(r0, ROWS_PER_TILE)])
    wait_idx(0)
    issue_gathers(0, 0)
    plsc.subcore_barrier()

    iota = lax.broadcasted_iota(jnp.int32, (16,), 0)
    mask_hi = jnp.where(iota >= 8, 1.0, 0.0).astype(jnp.float32)

    def bcast(w, lane):
        return lax.gather(
            w, jnp.full((16, 1), lane, jnp.int32),
            dimension_numbers=lax.GatherDimensionNumbers(
                offset_dims=(), collapsed_slice_dims=(0,),
                start_index_map=(0,)),
            slice_sizes=(1,),
            mode=lax.GatherScatterMode.PROMISE_IN_BOUNDS)

    def compute_chunk(b):
        zbuf, edbuf, msgbuf = zbufs[b], edbufs[b], msgbufs[b]

        @plsc.parallel_loop(0, CHUNK, unroll=8)
        def _(j):
            ea, _ = plsc.unpack(zbuf[j, pl.ds(HD, 32)],
                                format=plsc.PackFormat.INTERLEAVED)
            sm = ea + edbuf[j]
            sm = jnp.where(sm > 0, sm, 0.2 * sm)
            w = jnp.exp(sm) * mask_hi
            for k in range(4):
                za, zb = plsc.unpack(zbuf[j, pl.ds(32 * k, 32)],
                                     format=plsc.PackFormat.INTERLEAVED)
                msgbuf[j, pl.ds(32 * k, 16)] = bcast(w, 8 + 2 * k) * za
                msgbuf[j, pl.ds(32 * k + 16, 16)] = bcast(w, 9 + 2 * k) * zb
            msgbuf[j, pl.ds(HD, 16)] = w

    def quad_body(q, carry):
        for b4 in range(4):
            ci = 4 * q + b4
            b = b4 % 2
            wait_gathers(b)

            wait_idx(1 - b)
            issue_gathers(1 - b, (b4 + 1) % 4)

            @pl.when(ci >= 2)
            def _():
                wait_scatter(b)

            @pl.when(ci + 2 < NCHUNK)
            def _():
                issue_idx(ci + 2, b, (b4 + 2) % 4)

            compute_chunk(b)
            pltpu.async_copy(msgbufs[b], acc.at[dring[b4]], ssems[b],
                             add=True)
        return carry

    lax.fori_loop(0, (NCHUNK - 1) // 4, quad_body, 0)

    wait_gathers(0)
    wait_scatter(0)
    compute_chunk(0)
    pltpu.async_copy(msgbufs[0], acc.at[dring[(NCHUNK - 1) % 4]], ssems[0],
                     add=True)
    wait_scatter(1)
    wait_scatter(0)
    plsc.subcore_barrier()
    pltpu.sync_copy(acc.at[pl.ds(r0, ROWS_PER_TILE), pl.ds(0, HD)],
                    att_hbm.at[c, pl.ds(r0, ROWS_PER_TILE)])
    pltpu.sync_copy(acc.at[pl.ds(r0, ROWS_PER_TILE), pl.ds(HD, 16)],
                    den_hbm.at[c, pl.ds(r0, ROWS_PER_TILE)])


@jax.jit
def _edge(src, dst, ztab, ed16, zero):
    mesh = plsc.VectorSubcoreMesh(core_axis_name="c", subcore_axis_name="s")
    return pl.kernel(
        _edge_body,
        out_type=(jax.ShapeDtypeStruct((NUM_CORES, N, HD), jnp.float32),
                  jax.ShapeDtypeStruct((NUM_CORES, N, 16), jnp.float32)),
        mesh=mesh,
        scratch_types=[
            pltpu.VMEM((CHUNK,), jnp.int32),
            pltpu.VMEM((CHUNK,), jnp.int32),
            pltpu.VMEM((CHUNK,), jnp.int32),
            pltpu.VMEM((CHUNK,), jnp.int32),
            pltpu.VMEM((CHUNK,), jnp.int32),
            pltpu.VMEM((CHUNK,), jnp.int32),
            pltpu.VMEM((CHUNK, ZT_W), jnp.bfloat16),
            pltpu.VMEM((CHUNK, ZT_W), jnp.bfloat16),
            pltpu.VMEM((CHUNK, 16), jnp.float32),
            pltpu.VMEM((CHUNK, 16), jnp.float32),
            pltpu.VMEM((CHUNK, ACC_W), jnp.float32),
            pltpu.VMEM((CHUNK, ACC_W), jnp.float32),
            pltpu.VMEM_SHARED((N, ACC_W), jnp.float32),
            pltpu.SemaphoreType.DMA,
            pltpu.SemaphoreType.DMA,
            pltpu.SemaphoreType.DMA,
            pltpu.SemaphoreType.DMA,
            pltpu.SemaphoreType.DMA,
            pltpu.SemaphoreType.DMA,
        ],
        compiler_params=pltpu.CompilerParams(
            use_tc_tiling_on_sc=False, needs_layout_passes=False),
    )(src, dst, ztab, ed16, zero)


def _final_body(att_ref, den_ref, o_ref, r8_ref, out_ref):
    a = att_ref[0] + att_ref[1]
    d = den_ref[0] + den_ref[1]
    den = d[:, 8:16]
    denrep = jnp.dot(den, r8_ref[...], preferred_element_type=jnp.float32,
                     precision=lax.Precision.HIGHEST)
    x = a / (denrep + 1e-9)
    out_ref[...] = o_ref[...] + jnp.where(
        x > 0, x, jnp.exp(jnp.minimum(x, 0.0)) - 1.0)


_FIN_BLK = 1000


@jax.jit
def _final(att, den, o, r8):
    return pl.pallas_call(
        _final_body,
        grid=(N // _FIN_BLK,),
        in_specs=[
            pl.BlockSpec((NUM_CORES, _FIN_BLK, HD), lambda i: (0, i, 0)),
            pl.BlockSpec((NUM_CORES, _FIN_BLK, 16), lambda i: (0, i, 0)),
            pl.BlockSpec((_FIN_BLK, HD), lambda i: (i, 0)),
            pl.BlockSpec((H, HD), lambda i: (0, 0)),
        ],
        out_specs=pl.BlockSpec((_FIN_BLK, HD), lambda i: (i, 0)),
        out_shape=jax.ShapeDtypeStruct((N, HD), jnp.float32),
    )(att, den, o, r8)


def kernel(edge_index, o, h, W, a_src, a_dst):
    src = edge_index[0].astype(jnp.int32)
    dst = edge_index[1].astype(jnp.int32)
    cols = jnp.arange(HD, dtype=jnp.int32)
    perm = 32 * (cols // 32) + (cols % 2) * 16 + (cols % 32) // 2
    wf = W.reshape(D_IN, HD).astype(jnp.float32)[:, perm]
    af = a_src.reshape(HD)
    as32 = jnp.zeros((HD, 32), jnp.float32).at[
        cols, 16 + 2 * (perm // D_H)].set(af[perm])
    ad16 = jnp.zeros((HD, 16), jnp.float32).at[cols, 8 + cols // D_H].set(
        a_dst.reshape(HD))
    ad16 = ad16[perm, :]
    r8 = jnp.zeros((H, HD), jnp.float32).at[cols // D_H, cols].set(1.0)

    ztab, ed16 = _proj(h, wf, as32, ad16)
    zero = jnp.zeros((ROWS_PER_TILE, ACC_W), jnp.float32)
    att, den = _edge(src, dst, ztab, ed16, zero)
    return _final(att, den, o, r8)
